# baseline (device time: 129641 ns/iter reference)
import jax
import jax.numpy as jnp
from jax import lax
from jax.experimental import pallas as pl
from jax.experimental.pallas import tpu as pltpu

N_DEV = 4

S_D1, S_D3, S_D2A, S_D2B = 0, 1, 2, 3
C_D1, C_D3, C_D2 = 0, 1, 2


def kernel(x, w_mat):
    m_glob, k_loc = x.shape
    k, n = w_mat.shape
    m_loc = m_glob // N_DEV
    half = m_loc // 2

    def body(x_ref, w_ref, out_ref, comm_ref, blk_send, blk_recv,
             w_buf, w_sems, amax_ref, amax_send, amax_recv):
        my = lax.axis_index("i")

        barrier = pltpu.get_barrier_semaphore()
        for d in range(1, N_DEV):
            peer = lax.rem(my + d, N_DEV)
            pl.semaphore_signal(barrier, inc=1, device_id=(peer,),
                                device_id_type=pl.DeviceIdType.MESH)
        pl.semaphore_wait(barrier, N_DEV - 1)

        def rdma(src, dst, sem_idx, tgt):
            r = pltpu.make_async_remote_copy(
                src_ref=src, dst_ref=dst,
                send_sem=blk_send.at[sem_idx],
                recv_sem=blk_recv.at[sem_idx],
                device_id=(tgt,),
                device_id_type=pl.DeviceIdType.MESH,
            )
            r.start()
            return r

        tgt2 = lax.rem(my + 2, N_DEV)
        tgt1 = lax.rem(my + 1, N_DEV)
        tgt3 = lax.rem(my + 3, N_DEV)
        sends = [
            rdma(x_ref.at[pl.ds(tgt2 * m_loc, half), :],
                 comm_ref.at[C_D2, pl.ds(0, half), :], S_D2A, tgt2),
            rdma(x_ref.at[pl.ds(tgt2 * m_loc + half, half), :],
                 comm_ref.at[C_D2, pl.ds(half, half), :], S_D2B, tgt2),
            rdma(x_ref.at[pl.ds(tgt1 * m_loc, m_loc), :],
                 comm_ref.at[C_D1], S_D1, tgt1),
            rdma(x_ref.at[pl.ds(tgt3 * m_loc, m_loc), :],
                 comm_ref.at[C_D3], S_D3, tgt3),
        ]
        recv_d2a, recv_d2b, recv_d1, recv_d3 = sends

        w_srcs = [my, lax.rem(my + 3, N_DEV), tgt1, tgt2]

        def w_block(step):
            c = pltpu.make_async_copy(
                w_ref.at[pl.ds(w_srcs[step] * k_loc, k_loc), :],
                w_buf.at[step % 2],
                w_sems.at[step % 2],
            )
            c.start()
            return c

        w_copies = [w_block(0), w_block(1)]

        w_copies[0].wait()
        out_ref[...] = jnp.dot(
            x_ref[pl.ds(my * m_loc, m_loc), :],
            w_buf[0],
            preferred_element_type=jnp.float32,
        )
        w_copies.append(w_block(2))

        recv_d1.wait_recv()
        w_copies[1].wait()
        out_ref[...] += jnp.dot(comm_ref[C_D1], w_buf[1],
                                preferred_element_type=jnp.float32)
        w_copies.append(w_block(3))

        recv_d3.wait_recv()
        w_copies[2].wait()
        out_ref[...] += jnp.dot(comm_ref[C_D3], w_buf[0],
                                preferred_element_type=jnp.float32)

        recv_d2a.wait_recv()
        w_copies[3].wait()
        out_ref[pl.ds(0, half), :] += jnp.dot(
            comm_ref[C_D2, pl.ds(0, half), :], w_buf[1],
            preferred_element_type=jnp.float32)
        amax_a = jnp.max(jnp.abs(out_ref[pl.ds(0, half), :]))

        recv_d2b.wait_recv()
        out_ref[pl.ds(half, half), :] += jnp.dot(
            comm_ref[C_D2, pl.ds(half, half), :], w_buf[1],
            preferred_element_type=jnp.float32)
        amax_b = jnp.max(jnp.abs(out_ref[pl.ds(half, half), :]))

        amax_ref[0] = jnp.full((8, 128), jnp.maximum(amax_a, amax_b),
                               jnp.float32)
        amax_sends = []
        for d in range(1, N_DEV):
            tgt = lax.rem(my + d, N_DEV)
            r = pltpu.make_async_remote_copy(
                src_ref=amax_ref.at[0],
                dst_ref=amax_ref.at[d],
                send_sem=amax_send.at[d - 1],
                recv_sem=amax_recv.at[d - 1],
                device_id=(tgt,),
                device_id_type=pl.DeviceIdType.MESH,
            )
            r.start()
            amax_sends.append(r)
        for r in amax_sends:
            r.wait_recv()

        gmax = jnp.max(amax_ref[...])
        scale = gmax / 448.0
        q = jnp.clip(out_ref[...] / scale, -448.0, 448.0)
        out_ref[...] = q.astype(jnp.float8_e4m3fn).astype(jnp.float32) * scale

        for r in sends + amax_sends:
            r.wait_send()

    return pl.pallas_call(
        body,
        out_shape=jax.ShapeDtypeStruct((m_loc, n), jnp.float32),
        in_specs=[
            pl.BlockSpec(memory_space=pltpu.VMEM),
            pl.BlockSpec(memory_space=pl.ANY),
        ],
        out_specs=pl.BlockSpec(memory_space=pltpu.VMEM),
        scratch_shapes=[
            pltpu.VMEM((3, m_loc, k_loc), jnp.float32),
            pltpu.SemaphoreType.DMA((4,)),
            pltpu.SemaphoreType.DMA((4,)),
            pltpu.VMEM((2, k_loc, n), jnp.float32),
            pltpu.SemaphoreType.DMA((2,)),
            pltpu.VMEM((N_DEV, 8, 128), jnp.float32),
            pltpu.SemaphoreType.DMA((N_DEV - 1,)),
            pltpu.SemaphoreType.DMA((N_DEV - 1,)),
        ],
        compiler_params=pltpu.CompilerParams(
            collective_id=0,
            vmem_limit_bytes=63 * 1024 * 1024,
        ),
    )(x, w_mat)


# device time: 129513 ns/iter; 1.0010x vs baseline; 1.0010x over previous
import jax
import jax.numpy as jnp
from jax import lax
from jax.experimental import pallas as pl
from jax.experimental.pallas import tpu as pltpu

N_DEV = 4

S_D1, S_D3, S_D2A, S_D2B = 0, 1, 2, 3
C_D1, C_D3, C_D2 = 0, 1, 2


def kernel(x, w_mat):
    m_glob, k_loc = x.shape
    k, n = w_mat.shape
    m_loc = m_glob // N_DEV
    tail_rows = 128
    bulk = m_loc - tail_rows

    def body(x_ref, w_ref, out_ref, comm_ref, blk_send, blk_recv,
             w_buf, w_sems, amax_ref, amax_send, amax_recv):
        my = lax.axis_index("i")

        barrier = pltpu.get_barrier_semaphore()
        for d in range(1, N_DEV):
            peer = lax.rem(my + d, N_DEV)
            pl.semaphore_signal(barrier, inc=1, device_id=(peer,),
                                device_id_type=pl.DeviceIdType.MESH)
        pl.semaphore_wait(barrier, N_DEV - 1)

        def rdma(src, dst, sem_idx, tgt):
            r = pltpu.make_async_remote_copy(
                src_ref=src, dst_ref=dst,
                send_sem=blk_send.at[sem_idx],
                recv_sem=blk_recv.at[sem_idx],
                device_id=(tgt,),
                device_id_type=pl.DeviceIdType.MESH,
            )
            r.start()
            return r

        tgt2 = lax.rem(my + 2, N_DEV)
        tgt1 = lax.rem(my + 1, N_DEV)
        tgt3 = lax.rem(my + 3, N_DEV)
        sends = [
            rdma(x_ref.at[pl.ds(tgt2 * m_loc, bulk), :],
                 comm_ref.at[C_D2, pl.ds(0, bulk), :], S_D2A, tgt2),
            rdma(x_ref.at[pl.ds(tgt2 * m_loc + bulk, tail_rows), :],
                 comm_ref.at[C_D2, pl.ds(bulk, tail_rows), :], S_D2B, tgt2),
            rdma(x_ref.at[pl.ds(tgt1 * m_loc, m_loc), :],
                 comm_ref.at[C_D1], S_D1, tgt1),
            rdma(x_ref.at[pl.ds(tgt3 * m_loc, m_loc), :],
                 comm_ref.at[C_D3], S_D3, tgt3),
        ]
        recv_d2a, recv_d2b, recv_d1, recv_d3 = sends

        w_srcs = [my, lax.rem(my + 3, N_DEV), tgt1, tgt2]

        def w_block(step):
            c = pltpu.make_async_copy(
                w_ref.at[pl.ds(w_srcs[step] * k_loc, k_loc), :],
                w_buf.at[step % 2],
                w_sems.at[step % 2],
            )
            c.start()
            return c

        w_copies = [w_block(0), w_block(1)]

        w_copies[0].wait()
        out_ref[...] = jnp.dot(
            x_ref[pl.ds(my * m_loc, m_loc), :],
            w_buf[0],
            preferred_element_type=jnp.float32,
        )
        w_copies.append(w_block(2))

        recv_d1.wait_recv()
        w_copies[1].wait()
        out_ref[...] += jnp.dot(comm_ref[C_D1], w_buf[1],
                                preferred_element_type=jnp.float32)
        w_copies.append(w_block(3))

        recv_d3.wait_recv()
        w_copies[2].wait()
        out_ref[...] += jnp.dot(comm_ref[C_D3], w_buf[0],
                                preferred_element_type=jnp.float32)

        recv_d2a.wait_recv()
        w_copies[3].wait()
        out_ref[pl.ds(0, bulk), :] += jnp.dot(
            comm_ref[C_D2, pl.ds(0, bulk), :], w_buf[1],
            preferred_element_type=jnp.float32)
        amax_a = jnp.max(jnp.abs(out_ref[pl.ds(0, bulk), :]))

        recv_d2b.wait_recv()
        out_ref[pl.ds(bulk, tail_rows), :] += jnp.dot(
            comm_ref[C_D2, pl.ds(bulk, tail_rows), :], w_buf[1],
            preferred_element_type=jnp.float32)
        amax_b = jnp.max(jnp.abs(out_ref[pl.ds(bulk, tail_rows), :]))

        amax_ref[0] = jnp.full((8, 128), jnp.maximum(amax_a, amax_b),
                               jnp.float32)
        amax_sends = []
        for d in range(1, N_DEV):
            tgt = lax.rem(my + d, N_DEV)
            r = pltpu.make_async_remote_copy(
                src_ref=amax_ref.at[0],
                dst_ref=amax_ref.at[d],
                send_sem=amax_send.at[d - 1],
                recv_sem=amax_recv.at[d - 1],
                device_id=(tgt,),
                device_id_type=pl.DeviceIdType.MESH,
            )
            r.start()
            amax_sends.append(r)
        for r in amax_sends:
            r.wait_recv()

        gmax = jnp.max(amax_ref[...])
        scale = gmax / 448.0
        inv = 448.0 / gmax
        q = (out_ref[...] * inv).astype(jnp.float8_e4m3fn)
        out_ref[...] = q.astype(jnp.float32) * scale

        for r in sends + amax_sends:
            r.wait_send()

    return pl.pallas_call(
        body,
        out_shape=jax.ShapeDtypeStruct((m_loc, n), jnp.float32),
        in_specs=[
            pl.BlockSpec(memory_space=pltpu.VMEM),
            pl.BlockSpec(memory_space=pl.ANY),
        ],
        out_specs=pl.BlockSpec(memory_space=pltpu.VMEM),
        scratch_shapes=[
            pltpu.VMEM((3, m_loc, k_loc), jnp.float32),
            pltpu.SemaphoreType.DMA((4,)),
            pltpu.SemaphoreType.DMA((4,)),
            pltpu.VMEM((2, k_loc, n), jnp.float32),
            pltpu.SemaphoreType.DMA((2,)),
            pltpu.VMEM((N_DEV, 8, 128), jnp.float32),
            pltpu.SemaphoreType.DMA((N_DEV - 1,)),
            pltpu.SemaphoreType.DMA((N_DEV - 1,)),
        ],
        compiler_params=pltpu.CompilerParams(
            collective_id=0,
            vmem_limit_bytes=63 * 1024 * 1024,
        ),
    )(x, w_mat)


# device time: 111736 ns/iter; 1.1602x vs baseline; 1.1591x over previous
import jax
import jax.numpy as jnp
from jax import lax
from jax.experimental import pallas as pl
from jax.experimental.pallas import tpu as pltpu

N_DEV = 4
N_CHUNK = 4
F_D1, F_D3, F_D2 = 0, 1, 2
FLOW_DIST = {F_D1: 1, F_D3: 3, F_D2: 2}


def kernel(x, w_mat):
    m_glob, k_loc = x.shape
    k, n = w_mat.shape
    m_loc = m_glob // N_DEV
    rows = m_loc // N_CHUNK

    def body(x_ref, w_ref, out_ref, comm_ref, blk_send, blk_recv,
             x_loc, x_sem, w_buf, w_sems, amax_ref, amax_send, amax_recv):
        my = lax.axis_index("i")

        barrier = pltpu.get_barrier_semaphore()
        for d in range(1, N_DEV):
            peer = lax.rem(my + d, N_DEV)
            pl.semaphore_signal(barrier, inc=1, device_id=(peer,),
                                device_id_type=pl.DeviceIdType.MESH)
        pl.semaphore_wait(barrier, N_DEV - 1)

        sends = {}
        for f, dist in FLOW_DIST.items():
            tgt = lax.rem(my + dist, N_DEV)
            for c in range(N_CHUNK):
                sem = f * N_CHUNK + c
                r = pltpu.make_async_remote_copy(
                    src_ref=x_ref.at[pl.ds(tgt * m_loc + c * rows, rows), :],
                    dst_ref=comm_ref.at[f, pl.ds(c * rows, rows), :],
                    send_sem=blk_send.at[sem],
                    recv_sem=blk_recv.at[sem],
                    device_id=(tgt,),
                    device_id_type=pl.DeviceIdType.MESH,
                )
                r.start()
                sends[(f, c)] = r

        x_copy = pltpu.make_async_copy(
            x_ref.at[pl.ds(my * m_loc, m_loc), :], x_loc, x_sem)
        x_copy.start()

        w_srcs = {
            0: my,
            1: lax.rem(my + N_DEV - 1, N_DEV),
            2: lax.rem(my + 1, N_DEV),
        }

        def w_copy(slot, src):
            c = pltpu.make_async_copy(
                w_ref.at[pl.ds(src * k_loc, k_loc), :],
                w_buf.at[slot], w_sems.at[slot])
            c.start()
            return c

        w_copies = [w_copy(s, w_srcs[s]) for s in range(3)]

        x_copy.wait()
        w_copies[0].wait()
        out_ref[...] = jnp.dot(x_loc[...], w_buf[0],
                               preferred_element_type=jnp.float32)
        w_d2 = w_copy(0, lax.rem(my + 2, N_DEV))
        w_copies[1].wait()
        w_copies[2].wait()
        w_d2.wait()
        w_slot = {F_D1: 1, F_D3: 2, F_D2: 0}

        amax = None
        for c in range(N_CHUNK):
            rs = pl.ds(c * rows, rows)
            for f in (F_D1, F_D3, F_D2):
                sends[(f, c)].wait_recv()
                out_ref[rs, :] += jnp.dot(
                    comm_ref[f, rs, :], w_buf[w_slot[f]],
                    preferred_element_type=jnp.float32)
            a = jnp.max(jnp.abs(out_ref[rs, :]))
            amax = a if amax is None else jnp.maximum(amax, a)

        amax_ref[0] = jnp.full((8, 128), amax, jnp.float32)
        amax_sends = []
        for d in range(1, N_DEV):
            tgt = lax.rem(my + d, N_DEV)
            r = pltpu.make_async_remote_copy(
                src_ref=amax_ref.at[0],
                dst_ref=amax_ref.at[d],
                send_sem=amax_send.at[d - 1],
                recv_sem=amax_recv.at[d - 1],
                device_id=(tgt,),
                device_id_type=pl.DeviceIdType.MESH,
            )
            r.start()
            amax_sends.append(r)
        for r in amax_sends:
            r.wait_recv()

        gmax = jnp.max(amax_ref[...])
        scale = gmax / 448.0
        inv = 448.0 / gmax
        for c in range(N_CHUNK):
            rs = pl.ds(c * rows, rows)
            q = (out_ref[rs, :] * inv).astype(jnp.float8_e4m3fn)
            out_ref[rs, :] = q.astype(jnp.float32) * scale

        for r in sends.values():
            r.wait_send()
        for r in amax_sends:
            r.wait_send()

    return pl.pallas_call(
        body,
        out_shape=jax.ShapeDtypeStruct((m_loc, n), jnp.float32),
        in_specs=[
            pl.BlockSpec(memory_space=pl.ANY),
            pl.BlockSpec(memory_space=pl.ANY),
        ],
        out_specs=pl.BlockSpec(memory_space=pltpu.VMEM),
        scratch_shapes=[
            pltpu.VMEM((3, m_loc, k_loc), jnp.float32),
            pltpu.SemaphoreType.DMA((3 * N_CHUNK,)),
            pltpu.SemaphoreType.DMA((3 * N_CHUNK,)),
            pltpu.VMEM((m_loc, k_loc), jnp.float32),
            pltpu.SemaphoreType.DMA,
            pltpu.VMEM((3, k_loc, n), jnp.float32),
            pltpu.SemaphoreType.DMA((3,)),
            pltpu.VMEM((N_DEV, 8, 128), jnp.float32),
            pltpu.SemaphoreType.DMA((N_DEV - 1,)),
            pltpu.SemaphoreType.DMA((N_DEV - 1,)),
        ],
        compiler_params=pltpu.CompilerParams(
            collective_id=0,
            vmem_limit_bytes=63 * 1024 * 1024,
        ),
    )(x, w_mat)


# device time: 111273 ns/iter; 1.1651x vs baseline; 1.0042x over previous
import jax
import jax.numpy as jnp
from jax import lax
from jax.experimental import pallas as pl
from jax.experimental.pallas import tpu as pltpu

N_DEV = 4
N_CHUNK = 8
F_D1, F_D3, F_D2 = 0, 1, 2
FLOW_DIST = {F_D1: 1, F_D3: 3, F_D2: 2}


def kernel(x, w_mat):
    m_glob, k_loc = x.shape
    k, n = w_mat.shape
    m_loc = m_glob // N_DEV
    rows = m_loc // N_CHUNK

    def body(x_ref, w_ref, out_ref, comm_ref, blk_send, blk_recv,
             x_loc, x_sem, w_buf, w_sems, amax_ref, amax_send, amax_recv):
        my = lax.axis_index("i")

        barrier = pltpu.get_barrier_semaphore()
        for d in range(1, N_DEV):
            peer = lax.rem(my + d, N_DEV)
            pl.semaphore_signal(barrier, inc=1, device_id=(peer,),
                                device_id_type=pl.DeviceIdType.MESH)
        pl.semaphore_wait(barrier, N_DEV - 1)

        sends = {}
        for f, dist in FLOW_DIST.items():
            tgt = lax.rem(my + dist, N_DEV)
            for c in range(N_CHUNK):
                sem = f * N_CHUNK + c
                r = pltpu.make_async_remote_copy(
                    src_ref=x_ref.at[pl.ds(tgt * m_loc + c * rows, rows), :],
                    dst_ref=comm_ref.at[f, pl.ds(c * rows, rows), :],
                    send_sem=blk_send.at[sem],
                    recv_sem=blk_recv.at[sem],
                    device_id=(tgt,),
                    device_id_type=pl.DeviceIdType.MESH,
                )
                r.start()
                sends[(f, c)] = r

        x_copy = pltpu.make_async_copy(
            x_ref.at[pl.ds(my * m_loc, m_loc), :], x_loc, x_sem)
        x_copy.start()

        w_srcs = {
            0: my,
            1: lax.rem(my + N_DEV - 1, N_DEV),
            2: lax.rem(my + 1, N_DEV),
        }

        def w_copy(slot, src):
            c = pltpu.make_async_copy(
                w_ref.at[pl.ds(src * k_loc, k_loc), :],
                w_buf.at[slot], w_sems.at[slot])
            c.start()
            return c

        w_copies = [w_copy(s, w_srcs[s]) for s in range(3)]

        x_copy.wait()
        w_copies[0].wait()
        out_ref[...] = jnp.dot(x_loc[...], w_buf[0],
                               preferred_element_type=jnp.float32)
        w_d2 = w_copy(0, lax.rem(my + 2, N_DEV))
        w_copies[1].wait()
        w_copies[2].wait()
        w_d2.wait()
        w_slot = {F_D1: 1, F_D3: 2, F_D2: 0}

        amax = None
        for c in range(N_CHUNK):
            rs = pl.ds(c * rows, rows)
            for f in (F_D1, F_D3, F_D2):
                sends[(f, c)].wait_recv()
                out_ref[rs, :] += jnp.dot(
                    comm_ref[f, rs, :], w_buf[w_slot[f]],
                    preferred_element_type=jnp.float32)
            a = jnp.max(jnp.abs(out_ref[rs, :]))
            amax = a if amax is None else jnp.maximum(amax, a)

        amax_ref[0] = jnp.full((8, 128), amax, jnp.float32)
        amax_sends = []
        for d in range(1, N_DEV):
            tgt = lax.rem(my + d, N_DEV)
            r = pltpu.make_async_remote_copy(
                src_ref=amax_ref.at[0],
                dst_ref=amax_ref.at[d],
                send_sem=amax_send.at[d - 1],
                recv_sem=amax_recv.at[d - 1],
                device_id=(tgt,),
                device_id_type=pl.DeviceIdType.MESH,
            )
            r.start()
            amax_sends.append(r)
        for r in amax_sends:
            r.wait_recv()

        gmax = jnp.max(amax_ref[...])
        scale = gmax / 448.0
        inv = 448.0 / gmax
        for c in range(N_CHUNK):
            rs = pl.ds(c * rows, rows)
            q = (out_ref[rs, :] * inv).astype(jnp.float8_e4m3fn)
            out_ref[rs, :] = q.astype(jnp.float32) * scale

        for r in sends.values():
            r.wait_send()
        for r in amax_sends:
            r.wait_send()

    return pl.pallas_call(
        body,
        out_shape=jax.ShapeDtypeStruct((m_loc, n), jnp.float32),
        in_specs=[
            pl.BlockSpec(memory_space=pl.ANY),
            pl.BlockSpec(memory_space=pl.ANY),
        ],
        out_specs=pl.BlockSpec(memory_space=pltpu.VMEM),
        scratch_shapes=[
            pltpu.VMEM((3, m_loc, k_loc), jnp.float32),
            pltpu.SemaphoreType.DMA((3 * N_CHUNK,)),
            pltpu.SemaphoreType.DMA((3 * N_CHUNK,)),
            pltpu.VMEM((m_loc, k_loc), jnp.float32),
            pltpu.SemaphoreType.DMA,
            pltpu.VMEM((3, k_loc, n), jnp.float32),
            pltpu.SemaphoreType.DMA((3,)),
            pltpu.VMEM((N_DEV, 8, 128), jnp.float32),
            pltpu.SemaphoreType.DMA((N_DEV - 1,)),
            pltpu.SemaphoreType.DMA((N_DEV - 1,)),
        ],
        compiler_params=pltpu.CompilerParams(
            collective_id=0,
            vmem_limit_bytes=63 * 1024 * 1024,
        ),
    )(x, w_mat)
